# baseline (speedup 1.0000x reference)
import jax
import jax.numpy as jnp
from jax.experimental import pallas as pl
from jax.experimental.pallas import tpu as pltpu
from jax.experimental.pallas import tpu_sc as plsc

_WINDOW = 512
_FB = 1024


def _gather_kernel(num_indices: int, value_dim: int, num_rows: int):
    mesh = plsc.VectorSubcoreMesh(core_axis_name="core", subcore_axis_name="subcore")

    @pl.kernel(
        out_type=jax.ShapeDtypeStruct((num_indices, 2 * value_dim), jnp.float32),
        mesh=mesh,
        compiler_params=pltpu.CompilerParams(use_tc_tiling_on_sc=False),
    )
    def kernel(w_hbm, i_hbm, o_hbm):
        def body(i_vmem, o_vmem):
            pltpu.sync_copy(w_hbm.at[i_vmem.at[0]], o_vmem)

        pltpu.emit_pipeline(
            body,
            grid=(num_indices // _WINDOW,),
            in_specs=[pl.BlockSpec((1, _WINDOW), index_map=lambda i: (0, i))],
            out_specs=[
                pl.BlockSpec((_WINDOW, value_dim), index_map=lambda i: (i, 0))
            ],
            core_axis_name=("core", "subcore"),
            dimension_semantics=(pltpu.PARALLEL,),
        )(i_hbm, o_hbm)

    return kernel


def _fmt_kernel(s: int, b: int, d: int):
    def body(x_ref, o_ref):
        o_ref[0] = x_ref[0][:, :d].T

    return pl.pallas_call(
        body,
        grid=(s, b // _FB),
        in_specs=[pl.BlockSpec((1, _FB, 2 * d), lambda i, j: (i, j, 0))],
        out_specs=pl.BlockSpec((1, d, _FB), lambda i, j: (i, 0, j)),
        out_shape=jax.ShapeDtypeStruct((s, d, b), jnp.float32),
    )


@jax.jit
def _run(input_, weight):
    b, s = input_.shape
    v, d = weight.shape
    n = b * s

    w_pad = jnp.concatenate([weight, jnp.zeros((v, d), jnp.float32)], axis=1)
    w_lin = w_pad.reshape(2 * v, d)

    i0 = input_.T.reshape(n).astype(jnp.int32)
    idx = i0 * 2
    flat = _gather_kernel(n, d, 2 * v)(w_lin, idx.reshape(1, n))

    x3 = flat.reshape(s, b, 2 * d)
    out_p = _fmt_kernel(s, b, d)(x3)
    return jnp.transpose(out_p, (2, 0, 1))


def kernel(input_, weight):
    return _run(input_, weight)


# speedup vs baseline: 1.4555x; 1.4555x over previous
import jax
import jax.numpy as jnp
from jax.experimental import pallas as pl
from jax.experimental.pallas import tpu as pltpu
from jax.experimental.pallas import tpu_sc as plsc

_WINDOW = 512
_FB = 1024


def _gather_kernel(num_indices: int, value_dim: int, num_rows: int):
    mesh = plsc.VectorSubcoreMesh(core_axis_name="core", subcore_axis_name="subcore")

    @pl.kernel(
        out_type=jax.ShapeDtypeStruct((num_indices, 2 * value_dim), jnp.float32),
        mesh=mesh,
        compiler_params=pltpu.CompilerParams(use_tc_tiling_on_sc=False),
    )
    def kernel(w_hbm, i_hbm, o_hbm):
        def body(i_vmem, o_vmem):
            pltpu.sync_copy(w_hbm.at[i_vmem.at[0]], o_vmem)

        pltpu.emit_pipeline(
            body,
            grid=(num_indices // _WINDOW,),
            in_specs=[pl.BlockSpec((1, _WINDOW), index_map=lambda i: (0, i))],
            out_specs=[
                pl.BlockSpec((_WINDOW, value_dim), index_map=lambda i: (i, 0))
            ],
            core_axis_name=("core", "subcore"),
            dimension_semantics=(pltpu.PARALLEL,),
        )(i_hbm, o_hbm)

    return kernel


def _fmt_kernel(s: int, b: int, d: int):
    def body(x_ref, o_ref):
        o_ref[0] = x_ref[0][:, :d].T

    return pl.pallas_call(
        body,
        grid=(s, b // _FB),
        in_specs=[pl.BlockSpec((1, _FB, 2 * d), lambda i, j: (i, j, 0))],
        out_specs=pl.BlockSpec((1, d, _FB), lambda i, j: (i, 0, j)),
        out_shape=jax.ShapeDtypeStruct((s, d, b), jnp.float32),
    )


@jax.jit
def _run(input_, weight):
    b, s = input_.shape
    v, d = weight.shape
    n = b * s

    w_pad = jnp.concatenate([weight, jnp.zeros((v, d), jnp.float32)], axis=1)
    w_lin = w_pad.reshape(2 * v, d)

    i0 = input_.reshape(n).astype(jnp.int32)
    idx = i0 * 2
    flat = _gather_kernel(n, d, 2 * v)(w_lin, idx.reshape(1, n))

    x3 = flat.reshape(b, s, 2 * d)
    return x3[:, :, :d]


def kernel(input_, weight):
    return _run(input_, weight)


# speedup vs baseline: 1.5536x; 1.0674x over previous
import jax
import jax.numpy as jnp
from jax import lax
from jax.experimental import pallas as pl
from jax.experimental.pallas import tpu as pltpu
from jax.experimental.pallas import tpu_sc as plsc

_WINDOW = 512
_CB = 1024
_DIMS = (((0,), (0,)), ((), ()))


def _xp(x, d):
    eye = jnp.eye(d, dtype=jnp.float32)
    return lax.dot_general(x, eye, _DIMS, preferred_element_type=jnp.float32)


def _prep_main(v: int, d: int, p: int, nlines: int):
    nb = p // _CB
    rclamp = (v - _CB) // _CB

    def body(l_ref, r_ref, o_ref):
        o_ref[...] = jnp.concatenate(
            [_xp(l_ref[...], d), _xp(r_ref[...], d)], axis=1
        )

    return pl.pallas_call(
        body,
        grid=(nb,),
        in_specs=[
            pl.BlockSpec((d, _CB), lambda j: (0, j)),
            pl.BlockSpec((d, _CB), lambda j: (0, jnp.minimum(j + nb, rclamp))),
        ],
        out_specs=pl.BlockSpec((_CB, 2 * d), lambda j: (j, 0)),
        out_shape=jax.ShapeDtypeStruct((nlines, 2 * d), jnp.float32),
    )


def _prep_tail(v: int, d: int, p: int, nlines: int, tail_start: int, ntb: int):
    tb0 = tail_start // (2 * d)
    lb0 = p // d

    def body(x_ref, w2_ref, o_ref):
        t = _xp(x_ref[...], d)
        o_ref[...] = jnp.concatenate([t[:d], t[d:]], axis=1)

    return pl.pallas_call(
        body,
        grid=(ntb,),
        in_specs=[
            pl.BlockSpec((d, 2 * d), lambda j: (0, tb0 + j)),
            pl.BlockSpec(memory_space=pl.ANY),
        ],
        out_specs=pl.BlockSpec((d, 2 * d), lambda j: (lb0 + j, 0)),
        out_shape=jax.ShapeDtypeStruct((nlines, 2 * d), jnp.float32),
        input_output_aliases={1: 0},
    )


def _gather_kernel(num_indices: int, value_dim: int, num_rows: int):
    mesh = plsc.VectorSubcoreMesh(core_axis_name="core", subcore_axis_name="subcore")

    @pl.kernel(
        out_type=jax.ShapeDtypeStruct((num_indices, 2 * value_dim), jnp.float32),
        mesh=mesh,
        compiler_params=pltpu.CompilerParams(use_tc_tiling_on_sc=False),
    )
    def kernel(w_hbm, i_hbm, o_hbm):
        def body(i_vmem, o_vmem):
            pltpu.sync_copy(w_hbm.at[i_vmem.at[0]], o_vmem)

        pltpu.emit_pipeline(
            body,
            grid=(num_indices // _WINDOW,),
            in_specs=[pl.BlockSpec((1, _WINDOW), index_map=lambda i: (0, i))],
            out_specs=[
                pl.BlockSpec((_WINDOW, value_dim), index_map=lambda i: (i, 0))
            ],
            core_axis_name=("core", "subcore"),
            dimension_semantics=(pltpu.PARALLEL,),
        )(i_hbm, o_hbm)

    return kernel


@jax.jit
def _run(input_, weight):
    b, s = input_.shape
    v, d = weight.shape
    n = b * s

    p = _CB * ((v // (2 * _CB)) + 2)
    tail_start = (v // _CB) * _CB
    ntb = -(-(v - tail_start) // (2 * d))
    nlines = p + ntb * d

    wt = weight.T
    w2 = _prep_main(v, d, p, nlines)(wt, wt)
    w2 = _prep_tail(v, d, p, nlines, tail_start, ntb)(wt, w2)
    w_lin = w2.reshape(2 * nlines, d)

    i0 = input_.reshape(n).astype(jnp.int32)
    u = i0 - tail_start
    tail_flat = 2 * (p + d * (u >> 7) + (u & (d - 1))) + ((u >> 6) & 1)
    idx = jnp.where(
        i0 < p, 2 * i0, jnp.where(i0 < tail_start, 2 * (i0 - p) + 1, tail_flat)
    )
    flat = _gather_kernel(n, d, 2 * nlines)(w_lin, idx.reshape(1, n))

    x3 = flat.reshape(b, s, 2 * d)
    return x3[:, :, :d]


def kernel(input_, weight):
    return _run(input_, weight)


# speedup vs baseline: 1.5833x; 1.0191x over previous
import jax
import jax.numpy as jnp
from jax import lax
from jax.experimental import pallas as pl
from jax.experimental.pallas import tpu as pltpu
from jax.experimental.pallas import tpu_sc as plsc

_WINDOW = 512
_CB = 1024
_DIMS = (((0,), (0,)), ((), ()))


def _xp(x, d):
    return x.T


def _prep_main(v: int, d: int, p: int, nlines: int):
    nb = p // _CB
    rclamp = (v - _CB) // _CB

    def body(l_ref, r_ref, o_ref):
        o_ref[...] = jnp.concatenate(
            [_xp(l_ref[...], d), _xp(r_ref[...], d)], axis=1
        )

    return pl.pallas_call(
        body,
        grid=(nb,),
        in_specs=[
            pl.BlockSpec((d, _CB), lambda j: (0, j)),
            pl.BlockSpec((d, _CB), lambda j: (0, jnp.minimum(j + nb, rclamp))),
        ],
        out_specs=pl.BlockSpec((_CB, 2 * d), lambda j: (j, 0)),
        out_shape=jax.ShapeDtypeStruct((nlines, 2 * d), jnp.float32),
    )


def _prep_tail(v: int, d: int, p: int, nlines: int, tail_start: int, ntb: int):
    tb0 = tail_start // (2 * d)
    lb0 = p // d

    def body(x_ref, w2_ref, o_ref):
        t = _xp(x_ref[...], d)
        o_ref[...] = jnp.concatenate([t[:d], t[d:]], axis=1)

    return pl.pallas_call(
        body,
        grid=(ntb,),
        in_specs=[
            pl.BlockSpec((d, 2 * d), lambda j: (0, tb0 + j)),
            pl.BlockSpec(memory_space=pl.ANY),
        ],
        out_specs=pl.BlockSpec((d, 2 * d), lambda j: (lb0 + j, 0)),
        out_shape=jax.ShapeDtypeStruct((nlines, 2 * d), jnp.float32),
        input_output_aliases={1: 0},
    )


def _gather_kernel(num_indices: int, value_dim: int, num_rows: int):
    mesh = plsc.VectorSubcoreMesh(core_axis_name="core", subcore_axis_name="subcore")

    @pl.kernel(
        out_type=jax.ShapeDtypeStruct((num_indices, 2 * value_dim), jnp.float32),
        mesh=mesh,
        compiler_params=pltpu.CompilerParams(use_tc_tiling_on_sc=False),
    )
    def kernel(w_hbm, i_hbm, o_hbm):
        def body(i_vmem, o_vmem):
            pltpu.sync_copy(w_hbm.at[i_vmem.at[0]], o_vmem)

        pltpu.emit_pipeline(
            body,
            grid=(num_indices // _WINDOW,),
            in_specs=[pl.BlockSpec((1, _WINDOW), index_map=lambda i: (0, i))],
            out_specs=[
                pl.BlockSpec((_WINDOW, value_dim), index_map=lambda i: (i, 0))
            ],
            core_axis_name=("core", "subcore"),
            dimension_semantics=(pltpu.PARALLEL,),
        )(i_hbm, o_hbm)

    return kernel


@jax.jit
def _run(input_, weight):
    b, s = input_.shape
    v, d = weight.shape
    n = b * s

    p = _CB * ((v // (2 * _CB)) + 2)
    tail_start = (v // _CB) * _CB
    ntb = -(-(v - tail_start) // (2 * d))
    nlines = p + ntb * d

    wt = weight.T
    w2 = _prep_main(v, d, p, nlines)(wt, wt)
    w2 = _prep_tail(v, d, p, nlines, tail_start, ntb)(wt, w2)
    w_lin = w2.reshape(2 * nlines, d)

    i0 = input_.reshape(n).astype(jnp.int32)
    u = i0 - tail_start
    tail_flat = 2 * (p + d * (u >> 7) + (u & (d - 1))) + ((u >> 6) & 1)
    idx = jnp.where(
        i0 < p, 2 * i0, jnp.where(i0 < tail_start, 2 * (i0 - p) + 1, tail_flat)
    )
    flat = _gather_kernel(n, d, 2 * nlines)(w_lin, idx.reshape(1, n))

    x3 = flat.reshape(b, s, 2 * d)
    return x3[:, :, :d]


def kernel(input_, weight):
    return _run(input_, weight)


# speedup vs baseline: 1.8601x; 1.1748x over previous
import jax
import jax.numpy as jnp
from jax import lax
from jax.experimental import pallas as pl
from jax.experimental.pallas import tpu as pltpu
from jax.experimental.pallas import tpu_sc as plsc

_WINDOW = 512
_CB = 2048
_DIMS = (((0,), (0,)), ((), ()))


def _xp(x, d):
    return x.T


def _prep_main(v: int, d: int, p: int, nlines: int):
    nb = p // _CB
    rclamp = (v - _CB) // _CB

    def body(l_ref, r_ref, o_ref):
        o_ref[...] = jnp.concatenate(
            [_xp(l_ref[...], d), _xp(r_ref[...], d)], axis=1
        )

    return pl.pallas_call(
        body,
        grid=(nb,),
        in_specs=[
            pl.BlockSpec((d, _CB), lambda j: (0, j)),
            pl.BlockSpec((d, _CB), lambda j: (0, jnp.minimum(j + nb, rclamp))),
        ],
        out_specs=pl.BlockSpec((_CB, 2 * d), lambda j: (j, 0)),
        out_shape=jax.ShapeDtypeStruct((nlines, 2 * d), jnp.float32),
    )


def _prep_tail(v: int, d: int, p: int, nlines: int, tail_start: int, ntb: int):
    tb0 = tail_start // (2 * d)
    lb0 = p // d

    def body(x_ref, w2_ref, o_ref):
        t = _xp(x_ref[...], d)
        o_ref[...] = jnp.concatenate([t[:d], t[d:]], axis=1)

    return pl.pallas_call(
        body,
        grid=(ntb,),
        in_specs=[
            pl.BlockSpec((d, 2 * d), lambda j: (0, tb0 + j)),
            pl.BlockSpec(memory_space=pl.ANY),
        ],
        out_specs=pl.BlockSpec((d, 2 * d), lambda j: (lb0 + j, 0)),
        out_shape=jax.ShapeDtypeStruct((nlines, 2 * d), jnp.float32),
        input_output_aliases={1: 0},
    )


def _gather_kernel(num_indices: int, value_dim: int, num_rows: int):
    mesh = plsc.VectorSubcoreMesh(core_axis_name="core", subcore_axis_name="subcore")

    @pl.kernel(
        out_type=jax.ShapeDtypeStruct((num_indices, 2 * value_dim), jnp.float32),
        mesh=mesh,
        compiler_params=pltpu.CompilerParams(use_tc_tiling_on_sc=False),
    )
    def kernel(w_hbm, i_hbm, o_hbm):
        def body(i_vmem, o_vmem):
            pltpu.sync_copy(w_hbm.at[i_vmem.at[0]], o_vmem)

        pltpu.emit_pipeline(
            body,
            grid=(num_indices // _WINDOW,),
            in_specs=[pl.BlockSpec((1, _WINDOW), index_map=lambda i: (0, i))],
            out_specs=[
                pl.BlockSpec((_WINDOW, value_dim), index_map=lambda i: (i, 0))
            ],
            core_axis_name=("core", "subcore"),
            dimension_semantics=(pltpu.PARALLEL,),
        )(i_hbm, o_hbm)

    return kernel


@jax.jit
def _run(input_, weight):
    b, s = input_.shape
    v, d = weight.shape
    n = b * s

    p = _CB * ((v // (2 * _CB)) + 2)
    tail_start = (v // _CB) * _CB
    ntb = -(-(v - tail_start) // (2 * d))
    nlines = p + ntb * d

    wt = weight.T
    w2 = _prep_main(v, d, p, nlines)(wt, wt)
    w2 = _prep_tail(v, d, p, nlines, tail_start, ntb)(wt, w2)
    w_lin = w2.reshape(2 * nlines, d)

    i0 = input_.reshape(n).astype(jnp.int32)
    u = i0 - tail_start
    tail_flat = 2 * (p + d * (u >> 7) + (u & (d - 1))) + ((u >> 6) & 1)
    idx = jnp.where(
        i0 < p, 2 * i0, jnp.where(i0 < tail_start, 2 * (i0 - p) + 1, tail_flat)
    )
    flat = _gather_kernel(n, d, 2 * nlines)(w_lin, idx.reshape(1, n))

    x3 = flat.reshape(b, s, 2 * d)
    return x3[:, :, :d]


def kernel(input_, weight):
    return _run(input_, weight)


# speedup vs baseline: 2.0713x; 1.1135x over previous
import jax
import jax.numpy as jnp
from jax import lax
from jax.experimental import pallas as pl
from jax.experimental.pallas import tpu as pltpu
from jax.experimental.pallas import tpu_sc as plsc

_WINDOW = 512
_CB = 4096
_DIMS = (((0,), (0,)), ((), ()))


def _xp(x, d):
    return x.T


def _prep_main(v: int, d: int, p: int, nlines: int):
    nb = p // _CB
    rclamp = (v - _CB) // _CB

    def body(l_ref, r_ref, o_ref):
        o_ref[...] = jnp.concatenate(
            [_xp(l_ref[...], d), _xp(r_ref[...], d)], axis=1
        )

    return pl.pallas_call(
        body,
        grid=(nb,),
        in_specs=[
            pl.BlockSpec((d, _CB), lambda j: (0, j)),
            pl.BlockSpec((d, _CB), lambda j: (0, jnp.minimum(j + nb, rclamp))),
        ],
        out_specs=pl.BlockSpec((_CB, 2 * d), lambda j: (j, 0)),
        out_shape=jax.ShapeDtypeStruct((nlines, 2 * d), jnp.float32),
    )


def _prep_tail(v: int, d: int, p: int, nlines: int, tail_start: int, ntb: int):
    tb0 = tail_start // (2 * d)
    lb0 = p // d

    def body(x_ref, w2_ref, o_ref):
        t = _xp(x_ref[...], d)
        o_ref[...] = jnp.concatenate([t[:d], t[d:]], axis=1)

    return pl.pallas_call(
        body,
        grid=(ntb,),
        in_specs=[
            pl.BlockSpec((d, 2 * d), lambda j: (0, tb0 + j)),
            pl.BlockSpec(memory_space=pl.ANY),
        ],
        out_specs=pl.BlockSpec((d, 2 * d), lambda j: (lb0 + j, 0)),
        out_shape=jax.ShapeDtypeStruct((nlines, 2 * d), jnp.float32),
        input_output_aliases={1: 0},
    )


def _gather_kernel(num_indices: int, value_dim: int, num_rows: int):
    mesh = plsc.VectorSubcoreMesh(core_axis_name="core", subcore_axis_name="subcore")

    @pl.kernel(
        out_type=jax.ShapeDtypeStruct((num_indices, 2 * value_dim), jnp.float32),
        mesh=mesh,
        compiler_params=pltpu.CompilerParams(use_tc_tiling_on_sc=False),
    )
    def kernel(w_hbm, i_hbm, o_hbm):
        def body(i_vmem, o_vmem):
            pltpu.sync_copy(w_hbm.at[i_vmem.at[0]], o_vmem)

        pltpu.emit_pipeline(
            body,
            grid=(num_indices // _WINDOW,),
            in_specs=[pl.BlockSpec((1, _WINDOW), index_map=lambda i: (0, i))],
            out_specs=[
                pl.BlockSpec((_WINDOW, value_dim), index_map=lambda i: (i, 0))
            ],
            core_axis_name=("core", "subcore"),
            dimension_semantics=(pltpu.PARALLEL,),
        )(i_hbm, o_hbm)

    return kernel


@jax.jit
def _run(input_, weight):
    b, s = input_.shape
    v, d = weight.shape
    n = b * s

    p = _CB * ((v // (2 * _CB)) + 2)
    tail_start = (v // _CB) * _CB
    ntb = -(-(v - tail_start) // (2 * d))
    nlines = p + ntb * d

    wt = weight.T
    w2 = _prep_main(v, d, p, nlines)(wt, wt)
    w2 = _prep_tail(v, d, p, nlines, tail_start, ntb)(wt, w2)
    w_lin = w2.reshape(2 * nlines, d)

    i0 = input_.reshape(n).astype(jnp.int32)
    u = i0 - tail_start
    tail_flat = 2 * (p + d * (u >> 7) + (u & (d - 1))) + ((u >> 6) & 1)
    idx = jnp.where(
        i0 < p, 2 * i0, jnp.where(i0 < tail_start, 2 * (i0 - p) + 1, tail_flat)
    )
    flat = _gather_kernel(n, d, 2 * nlines)(w_lin, idx.reshape(1, n))

    x3 = flat.reshape(b, s, 2 * d)
    return x3[:, :, :d]


def kernel(input_, weight):
    return _run(input_, weight)


# speedup vs baseline: 2.1801x; 1.0525x over previous
import jax
import jax.numpy as jnp
from jax import lax
from jax.experimental import pallas as pl
from jax.experimental.pallas import tpu as pltpu
from jax.experimental.pallas import tpu_sc as plsc

_WINDOW = 512
_CB = 8192
_DIMS = (((0,), (0,)), ((), ()))


def _xp(x, d):
    return x.T


def _prep_main(v: int, d: int, p: int, nlines: int):
    nb = p // _CB
    rclamp = (v - _CB) // _CB

    def body(l_ref, r_ref, o_ref):
        o_ref[...] = jnp.concatenate(
            [_xp(l_ref[...], d), _xp(r_ref[...], d)], axis=1
        )

    return pl.pallas_call(
        body,
        grid=(nb,),
        in_specs=[
            pl.BlockSpec((d, _CB), lambda j: (0, j)),
            pl.BlockSpec((d, _CB), lambda j: (0, jnp.minimum(j + nb, rclamp))),
        ],
        out_specs=pl.BlockSpec((_CB, 2 * d), lambda j: (j, 0)),
        out_shape=jax.ShapeDtypeStruct((nlines, 2 * d), jnp.float32),
    )


def _prep_tail(v: int, d: int, p: int, nlines: int, tail_start: int, ntb: int):
    tb0 = tail_start // (2 * d)
    lb0 = p // d

    def body(x_ref, w2_ref, o_ref):
        t = _xp(x_ref[...], d)
        o_ref[...] = jnp.concatenate([t[:d], t[d:]], axis=1)

    return pl.pallas_call(
        body,
        grid=(ntb,),
        in_specs=[
            pl.BlockSpec((d, 2 * d), lambda j: (0, tb0 + j)),
            pl.BlockSpec(memory_space=pl.ANY),
        ],
        out_specs=pl.BlockSpec((d, 2 * d), lambda j: (lb0 + j, 0)),
        out_shape=jax.ShapeDtypeStruct((nlines, 2 * d), jnp.float32),
        input_output_aliases={1: 0},
    )


def _gather_kernel(num_indices: int, value_dim: int, num_rows: int):
    mesh = plsc.VectorSubcoreMesh(core_axis_name="core", subcore_axis_name="subcore")

    @pl.kernel(
        out_type=jax.ShapeDtypeStruct((num_indices, 2 * value_dim), jnp.float32),
        mesh=mesh,
        compiler_params=pltpu.CompilerParams(use_tc_tiling_on_sc=False),
    )
    def kernel(w_hbm, i_hbm, o_hbm):
        def body(i_vmem, o_vmem):
            pltpu.sync_copy(w_hbm.at[i_vmem.at[0]], o_vmem)

        pltpu.emit_pipeline(
            body,
            grid=(num_indices // _WINDOW,),
            in_specs=[pl.BlockSpec((1, _WINDOW), index_map=lambda i: (0, i))],
            out_specs=[
                pl.BlockSpec((_WINDOW, value_dim), index_map=lambda i: (i, 0))
            ],
            core_axis_name=("core", "subcore"),
            dimension_semantics=(pltpu.PARALLEL,),
        )(i_hbm, o_hbm)

    return kernel


@jax.jit
def _run(input_, weight):
    b, s = input_.shape
    v, d = weight.shape
    n = b * s

    p = _CB * ((v // (2 * _CB)) + 2)
    tail_start = (v // _CB) * _CB
    ntb = -(-(v - tail_start) // (2 * d))
    nlines = p + ntb * d

    wt = weight.T
    w2 = _prep_main(v, d, p, nlines)(wt, wt)
    w2 = _prep_tail(v, d, p, nlines, tail_start, ntb)(wt, w2)
    w_lin = w2.reshape(2 * nlines, d)

    i0 = input_.reshape(n).astype(jnp.int32)
    u = i0 - tail_start
    tail_flat = 2 * (p + d * (u >> 7) + (u & (d - 1))) + ((u >> 6) & 1)
    idx = jnp.where(
        i0 < p, 2 * i0, jnp.where(i0 < tail_start, 2 * (i0 - p) + 1, tail_flat)
    )
    flat = _gather_kernel(n, d, 2 * nlines)(w_lin, idx.reshape(1, n))

    x3 = flat.reshape(b, s, 2 * d)
    return x3[:, :, :d]


def kernel(input_, weight):
    return _run(input_, weight)
